# baseline (speedup 1.0000x reference)
import jax
import jax.numpy as jnp
from jax import lax
from jax.experimental import pallas as pl
from jax.experimental.pallas import tpu as pltpu
from jax.experimental.pallas import tpu_sc as plsc

EDGE_IN = 128
NODE_OUT = 128
N_HE = 10000
N_INC = 320000

_NC = 2
_NS = 16
_NW = _NC * _NS
_SLICE_CFG = [(800, 32), (1200, 80), (3000, 120), (3000, 120),
              (1200, 80), (800, 32)]
_BLK = 1600


def _pe_body(he_ref, w1_ref, w2_ref, b_ref, pe1_ref, pe2_ref):
    he = he_ref[...]
    pe1_ref[...] = (
        jnp.dot(he, w1_ref[...], preferred_element_type=jnp.float32) + b_ref[...]
    )
    pe2_ref[...] = jnp.dot(he, w2_ref[...], preferred_element_type=jnp.float32)


def _fuse_body(hn_ref, g_ref, w3_ref, out_ref):
    acc = jnp.dot(hn_ref[...], w3_ref[...], preferred_element_type=jnp.float32)
    g = g_ref[...].reshape(_BLK, NODE_OUT)
    out_ref[...] = jnp.maximum(acc + g, 0.0)


def _make_gather_body(bpw, c):
    nchunk = bpw // c
    assert nchunk % 2 == 1

    def body(pe1_hbm, pe2_hbm, idx0_hbm, idx1_hbm, g_hbm,
             idx0_v, idx1_v, r1a, r1b, r2a, r2b, oa, ob,
             gsema, gsemb, wsema, wsemb):
        wid = lax.axis_index("s") * _NC + lax.axis_index("c")
        base = wid * bpw
        icp0 = pltpu.make_async_copy(idx0_hbm.at[pl.ds(base, bpw)], idx0_v,
                                     gsema)
        icp1 = pltpu.make_async_copy(idx1_hbm.at[pl.ds(base, bpw)], idx1_v,
                                     gsemb)
        icp0.start()
        icp1.start()
        icp0.wait()
        icp1.wait()

        r1 = (r1a, r1b)
        r2 = (r2a, r2b)
        ob_ = (oa, ob)
        gsem = (gsema, gsemb)
        wsem = (wsema, wsemb)

        def fire_gather(k, p):
            pltpu.make_async_copy(
                pe1_hbm.at[idx0_v.at[pl.ds(k * c, c)]], r1[p], gsem[p]).start()
            pltpu.make_async_copy(
                pe2_hbm.at[idx1_v.at[pl.ds(k * c, c)]], r2[p], gsem[p]).start()

        def wait_gather(p):
            pltpu.make_async_copy(
                pe1_hbm.at[idx0_v.at[pl.ds(0, c)]], r1[p], gsem[p]).wait()
            pltpu.make_async_copy(
                pe2_hbm.at[idx1_v.at[pl.ds(0, c)]], r2[p], gsem[p]).wait()

        def compute(p, q):
            def addrow(r, c2):
                rowoff = r * NODE_OUT
                for j in range(NODE_OUT // 16):
                    sl = pl.ds(j * 16, 16)
                    o = pl.multiple_of(rowoff + j * 16, 16)
                    ob_[q][pl.ds(o, 16)] = r1[p][r, sl] + r2[p][r, sl]
                return c2
            lax.fori_loop(0, c, addrow, 0, unroll=2)

        def fire_wb(k, q):
            pltpu.make_async_copy(
                ob_[q],
                g_hbm.at[pl.ds((base + k * c) * NODE_OUT, c * NODE_OUT)],
                wsem[q]).start()

        def wait_wb(q):
            pltpu.make_async_copy(
                ob_[q], g_hbm.at[pl.ds(base * NODE_OUT, c * NODE_OUT)],
                wsem[q]).wait()

        fire_gather(0, 0)

        def pair(m, carry):
            k0 = 2 * m
            fire_gather(k0 + 1, 1)
            wait_gather(0)

            @pl.when(m > 0)
            def _():
                wait_wb(0)
            compute(0, 0)
            fire_wb(k0, 0)
            fire_gather(k0 + 2, 0)
            wait_gather(1)

            @pl.when(m > 0)
            def _():
                wait_wb(1)
            compute(1, 1)
            fire_wb(k0 + 1, 1)
            return carry

        lax.fori_loop(0, (nchunk - 1) // 2, pair, 0)
        wait_gather(0)
        wait_wb(0)
        compute(0, 0)
        fire_wb(nchunk - 1, 0)
        wait_wb(1)
        wait_wb(0)

    return body


def _gather_sum(pe1, pe2, idx0_s, idx1_s, bpw, c):
    size = bpw * _NW
    mesh = plsc.VectorSubcoreMesh(
        core_axis_name="c", subcore_axis_name="s",
        num_cores=_NC, num_subcores=_NS)
    return pl.kernel(
        _make_gather_body(bpw, c),
        out_type=jax.ShapeDtypeStruct((size * NODE_OUT,), jnp.float32),
        mesh=mesh,
        scratch_types=(
            [pltpu.VMEM((bpw,), jnp.int32)] * 2
            + [pltpu.VMEM((c, NODE_OUT), jnp.float32)] * 4
            + [pltpu.VMEM((c * NODE_OUT,), jnp.float32)] * 2
            + [pltpu.SemaphoreType.DMA] * 4
        ),
    )(pe1, pe2, idx0_s, idx1_s)


def kernel(hyperedge, hyper_node, ve_affiliation, W, b):
    idx0 = ve_affiliation[0].astype(jnp.int32)
    idx1 = ve_affiliation[1].astype(jnp.int32)
    w1 = W[:EDGE_IN]
    w2 = W[EDGE_IN:2 * EDGE_IN]
    w3 = W[2 * EDGE_IN:]
    b2 = b.reshape(1, NODE_OUT)

    pe1, pe2 = pl.pallas_call(
        _pe_body,
        out_shape=(
            jax.ShapeDtypeStruct((N_HE, NODE_OUT), jnp.float32),
            jax.ShapeDtypeStruct((N_HE, NODE_OUT), jnp.float32),
        ),
    )(hyperedge, w1, w2, b2)

    starts = []
    acc = 0
    for bpw, c in _SLICE_CFG:
        starts.append(acc)
        acc += bpw * _NW

    gs = []
    for (bpw, c), start in zip(_SLICE_CFG, starts):
        size = bpw * _NW
        gs.append(_gather_sum(
            pe1, pe2,
            lax.slice_in_dim(idx0, start, start + size),
            lax.slice_in_dim(idx1, start, start + size),
            bpw, c))

    out = None
    for s, ((bpw, c), start) in enumerate(zip(_SLICE_CFG, starts)):
        size = bpw * _NW
        nblk = size // _BLK
        boff = start // _BLK
        hn_spec = pl.BlockSpec((_BLK, EDGE_IN),
                               lambda i, boff=boff: (i + boff, 0))
        out_spec = pl.BlockSpec((_BLK, NODE_OUT),
                                lambda i, boff=boff: (i + boff, 0))
        in_specs = [
            hn_spec,
            pl.BlockSpec((_BLK * NODE_OUT,), lambda i: (i,)),
            pl.BlockSpec((EDGE_IN, NODE_OUT), lambda i: (0, 0)),
        ]
        if s == 0:
            out = pl.pallas_call(
                _fuse_body,
                grid=(nblk,),
                in_specs=in_specs,
                out_specs=out_spec,
                out_shape=jax.ShapeDtypeStruct((N_INC, NODE_OUT), jnp.float32),
            )(hyper_node, gs[s], w3)
        else:
            def _fuse_acc_body(hn_ref, g_ref, w3_ref, prev_ref, out_ref):
                _fuse_body(hn_ref, g_ref, w3_ref, out_ref)

            out = pl.pallas_call(
                _fuse_acc_body,
                grid=(nblk,),
                in_specs=in_specs + [out_spec],
                out_specs=out_spec,
                out_shape=jax.ShapeDtypeStruct((N_INC, NODE_OUT), jnp.float32),
                input_output_aliases={3: 0},
            )(hyper_node, gs[s], w3, out)
    return out


# speedup vs baseline: 1.0226x; 1.0226x over previous
import jax
import jax.numpy as jnp
from jax import lax
from jax.experimental import pallas as pl
from jax.experimental.pallas import tpu as pltpu
from jax.experimental.pallas import tpu_sc as plsc

EDGE_IN = 128
NODE_OUT = 128
N_HE = 10000
N_INC = 320000

_NC = 2
_NS = 16
_NW = _NC * _NS
_SLICE_CFG = [(800, 32), (1200, 48), (2000, 80), (2000, 80), (2000, 80),
              (1200, 48), (800, 32)]
_BLK = 1600


def _pe_body(he_ref, w1_ref, w2_ref, b_ref, pe1_ref, pe2_ref):
    he = he_ref[...]
    pe1_ref[...] = (
        jnp.dot(he, w1_ref[...], preferred_element_type=jnp.float32) + b_ref[...]
    )
    pe2_ref[...] = jnp.dot(he, w2_ref[...], preferred_element_type=jnp.float32)


def _fuse_body(hn_ref, g_ref, w3_ref, out_ref):
    acc = jnp.dot(hn_ref[...], w3_ref[...], preferred_element_type=jnp.float32)
    g = g_ref[...].reshape(_BLK, NODE_OUT)
    out_ref[...] = jnp.maximum(acc + g, 0.0)


def _make_gather_body(bpw, c):
    nchunk = bpw // c
    assert nchunk % 2 == 1

    def body(pe1_hbm, pe2_hbm, idx0_hbm, idx1_hbm, g_hbm,
             idx0_v, idx1_v, r1a, r1b, r2a, r2b, oa, ob,
             gsema, gsemb, wsema, wsemb):
        wid = lax.axis_index("s") * _NC + lax.axis_index("c")
        base = wid * bpw
        icp0 = pltpu.make_async_copy(idx0_hbm.at[pl.ds(base, bpw)], idx0_v,
                                     gsema)
        icp1 = pltpu.make_async_copy(idx1_hbm.at[pl.ds(base, bpw)], idx1_v,
                                     gsemb)
        icp0.start()
        icp1.start()
        icp0.wait()
        icp1.wait()

        r1 = (r1a, r1b)
        r2 = (r2a, r2b)
        ob_ = (oa, ob)
        gsem = (gsema, gsemb)
        wsem = (wsema, wsemb)

        def fire_gather(k, p):
            pltpu.make_async_copy(
                pe1_hbm.at[idx0_v.at[pl.ds(k * c, c)]], r1[p], gsem[p]).start()
            pltpu.make_async_copy(
                pe2_hbm.at[idx1_v.at[pl.ds(k * c, c)]], r2[p], gsem[p]).start()

        def wait_gather(p):
            pltpu.make_async_copy(
                pe1_hbm.at[idx0_v.at[pl.ds(0, c)]], r1[p], gsem[p]).wait()
            pltpu.make_async_copy(
                pe2_hbm.at[idx1_v.at[pl.ds(0, c)]], r2[p], gsem[p]).wait()

        def compute(p, q):
            def addrow(r, c2):
                rowoff = r * NODE_OUT
                for j in range(NODE_OUT // 16):
                    sl = pl.ds(j * 16, 16)
                    o = pl.multiple_of(rowoff + j * 16, 16)
                    ob_[q][pl.ds(o, 16)] = r1[p][r, sl] + r2[p][r, sl]
                return c2
            lax.fori_loop(0, c, addrow, 0, unroll=2)

        def fire_wb(k, q):
            pltpu.make_async_copy(
                ob_[q],
                g_hbm.at[pl.ds((base + k * c) * NODE_OUT, c * NODE_OUT)],
                wsem[q]).start()

        def wait_wb(q):
            pltpu.make_async_copy(
                ob_[q], g_hbm.at[pl.ds(base * NODE_OUT, c * NODE_OUT)],
                wsem[q]).wait()

        fire_gather(0, 0)

        def pair(m, carry):
            k0 = 2 * m
            fire_gather(k0 + 1, 1)
            wait_gather(0)

            @pl.when(m > 0)
            def _():
                wait_wb(0)
            compute(0, 0)
            fire_wb(k0, 0)
            fire_gather(k0 + 2, 0)
            wait_gather(1)

            @pl.when(m > 0)
            def _():
                wait_wb(1)
            compute(1, 1)
            fire_wb(k0 + 1, 1)
            return carry

        lax.fori_loop(0, (nchunk - 1) // 2, pair, 0)
        wait_gather(0)
        wait_wb(0)
        compute(0, 0)
        fire_wb(nchunk - 1, 0)
        wait_wb(1)
        wait_wb(0)

    return body


def _gather_sum(pe1, pe2, idx0_s, idx1_s, bpw, c):
    size = bpw * _NW
    mesh = plsc.VectorSubcoreMesh(
        core_axis_name="c", subcore_axis_name="s",
        num_cores=_NC, num_subcores=_NS)
    return pl.kernel(
        _make_gather_body(bpw, c),
        out_type=jax.ShapeDtypeStruct((size * NODE_OUT,), jnp.float32),
        mesh=mesh,
        scratch_types=(
            [pltpu.VMEM((bpw,), jnp.int32)] * 2
            + [pltpu.VMEM((c, NODE_OUT), jnp.float32)] * 4
            + [pltpu.VMEM((c * NODE_OUT,), jnp.float32)] * 2
            + [pltpu.SemaphoreType.DMA] * 4
        ),
    )(pe1, pe2, idx0_s, idx1_s)


def kernel(hyperedge, hyper_node, ve_affiliation, W, b):
    idx0 = ve_affiliation[0].astype(jnp.int32)
    idx1 = ve_affiliation[1].astype(jnp.int32)
    w1 = W[:EDGE_IN]
    w2 = W[EDGE_IN:2 * EDGE_IN]
    w3 = W[2 * EDGE_IN:]
    b2 = b.reshape(1, NODE_OUT)

    pe1, pe2 = pl.pallas_call(
        _pe_body,
        out_shape=(
            jax.ShapeDtypeStruct((N_HE, NODE_OUT), jnp.float32),
            jax.ShapeDtypeStruct((N_HE, NODE_OUT), jnp.float32),
        ),
    )(hyperedge, w1, w2, b2)

    starts = []
    acc = 0
    for bpw, c in _SLICE_CFG:
        starts.append(acc)
        acc += bpw * _NW

    gs = []
    for (bpw, c), start in zip(_SLICE_CFG, starts):
        size = bpw * _NW
        gs.append(_gather_sum(
            pe1, pe2,
            lax.slice_in_dim(idx0, start, start + size),
            lax.slice_in_dim(idx1, start, start + size),
            bpw, c))

    out = None
    for s, ((bpw, c), start) in enumerate(zip(_SLICE_CFG, starts)):
        size = bpw * _NW
        nblk = size // _BLK
        boff = start // _BLK
        hn_spec = pl.BlockSpec((_BLK, EDGE_IN),
                               lambda i, boff=boff: (i + boff, 0))
        out_spec = pl.BlockSpec((_BLK, NODE_OUT),
                                lambda i, boff=boff: (i + boff, 0))
        in_specs = [
            hn_spec,
            pl.BlockSpec((_BLK * NODE_OUT,), lambda i: (i,)),
            pl.BlockSpec((EDGE_IN, NODE_OUT), lambda i: (0, 0)),
        ]
        if s == 0:
            out = pl.pallas_call(
                _fuse_body,
                grid=(nblk,),
                in_specs=in_specs,
                out_specs=out_spec,
                out_shape=jax.ShapeDtypeStruct((N_INC, NODE_OUT), jnp.float32),
            )(hyper_node, gs[s], w3)
        else:
            def _fuse_acc_body(hn_ref, g_ref, w3_ref, prev_ref, out_ref):
                _fuse_body(hn_ref, g_ref, w3_ref, out_ref)

            out = pl.pallas_call(
                _fuse_acc_body,
                grid=(nblk,),
                in_specs=in_specs + [out_spec],
                out_specs=out_spec,
                out_shape=jax.ShapeDtypeStruct((N_INC, NODE_OUT), jnp.float32),
                input_output_aliases={3: 0},
            )(hyper_node, gs[s], w3, out)
    return out
